# baseline (speedup 1.0000x reference)
import jax
import jax.numpy as jnp
from jax.experimental import pallas as pl
from jax.experimental.pallas import tpu as pltpu

N_HALF = 25000
D = 256
BC = 5000
NCH = N_HALF // BC
NC = 2 * NCH
NBUF = 4
NSUB = 5
BS = BC // NSUB


def _mm_kernel(ctx_hbm, smp_hbm, w_ref, b_ref, out_hbm, xbuf, obuf, in_sem, out_sem):
    def start_in(c, slot):
        j = c // 2

        @pl.when(c % 2 == 0)
        def _():
            pltpu.make_async_copy(
                ctx_hbm.at[pl.ds(j * BC, BC), :], xbuf.at[slot], in_sem.at[slot]
            ).start()

        @pl.when(c % 2 == 1)
        def _():
            pltpu.make_async_copy(
                smp_hbm.at[pl.ds(j * BC, BC), :], xbuf.at[slot], in_sem.at[slot]
            ).start()

    def wait_in(slot):
        pltpu.make_async_copy(
            ctx_hbm.at[pl.ds(0, BC), :], xbuf.at[slot], in_sem.at[slot]
        ).wait()

    def out_off(c):
        return (c % 2) * N_HALF + (c // 2) * BC

    def start_out_piece(c, slot, k):
        pltpu.make_async_copy(
            obuf.at[slot, pl.ds(k * BS, BS), :],
            out_hbm.at[pl.ds(out_off(c) + k * BS, BS), :],
            out_sem.at[slot],
        ).start()

    def wait_out(c, slot):
        for k in range(NSUB):
            pltpu.make_async_copy(
                obuf.at[slot, pl.ds(k * BS, BS), :],
                out_hbm.at[pl.ds(out_off(c) + k * BS, BS), :],
                out_sem.at[slot],
            ).wait()

    for s in range(NBUF):
        start_in(s, s)

    def body(c, carry):
        slot = jax.lax.rem(c, NBUF)

        @pl.when(c >= NBUF)
        def _():
            wait_out(c - NBUF, slot)

        wait_in(slot)
        for k in range(NSUB):
            obuf[slot, pl.ds(k * BS, BS), :] = (
                jnp.dot(
                    xbuf[slot, pl.ds(k * BS, BS), :],
                    w_ref[...],
                    preferred_element_type=jnp.float32,
                )
                + b_ref[...]
            )
            start_out_piece(c, slot, k)

        @pl.when(c + NBUF < NC)
        def _():
            start_in(c + NBUF, slot)

        return carry

    jax.lax.fori_loop(0, NC, body, 0)

    for k in range(NC - NBUF, NC):
        wait_out(k, k % NBUF)


def kernel(context, sample, W_proj, b_proj):
    b2d = b_proj.reshape(1, D)
    out = pl.pallas_call(
        _mm_kernel,
        in_specs=[
            pl.BlockSpec(memory_space=pl.ANY),
            pl.BlockSpec(memory_space=pl.ANY),
            pl.BlockSpec(memory_space=pltpu.VMEM),
            pl.BlockSpec(memory_space=pltpu.VMEM),
        ],
        out_specs=pl.BlockSpec(memory_space=pl.ANY),
        out_shape=jax.ShapeDtypeStruct((2 * N_HALF, D), jnp.float32),
        scratch_shapes=[
            pltpu.VMEM((NBUF, BC, D), jnp.float32),
            pltpu.VMEM((NBUF, BC, D), jnp.float32),
            pltpu.SemaphoreType.DMA((NBUF,)),
            pltpu.SemaphoreType.DMA((NBUF,)),
        ],
    )(context, sample, W_proj, b2d)
    return out


# speedup vs baseline: 1.0067x; 1.0067x over previous
import jax
import jax.numpy as jnp
from jax.experimental import pallas as pl
from jax.experimental.pallas import tpu as pltpu

N_HALF = 25000
D = 256
BC = 5000
NCH = N_HALF // BC
NC = 2 * NCH
NBUF = 4


def _mm_kernel(ctx_hbm, smp_hbm, w_ref, b_ref, out_hbm, xbuf, obuf, in_sem, out_sem):
    def start_in(c, slot):
        @pl.when(c < NCH)
        def _():
            pltpu.make_async_copy(
                ctx_hbm.at[pl.ds(c * BC, BC), :], xbuf.at[slot], in_sem.at[slot]
            ).start()

        @pl.when(c >= NCH)
        def _():
            pltpu.make_async_copy(
                smp_hbm.at[pl.ds((c - NCH) * BC, BC), :], xbuf.at[slot], in_sem.at[slot]
            ).start()

    def wait_in(slot):
        pltpu.make_async_copy(
            ctx_hbm.at[pl.ds(0, BC), :], xbuf.at[slot], in_sem.at[slot]
        ).wait()

    def out_off(c):
        return c * BC

    def start_out(c, slot):
        pltpu.make_async_copy(
            obuf.at[slot], out_hbm.at[pl.ds(out_off(c), BC), :], out_sem.at[slot]
        ).start()

    def wait_out(c, slot):
        pltpu.make_async_copy(
            obuf.at[slot], out_hbm.at[pl.ds(out_off(c), BC), :], out_sem.at[slot]
        ).wait()

    for s in range(NBUF):
        start_in(s, s)

    def body(c, carry):
        slot = jax.lax.rem(c, NBUF)

        @pl.when(c >= NBUF)
        def _():
            wait_out(c - NBUF, slot)

        wait_in(slot)
        obuf[slot] = (
            jnp.dot(xbuf[slot], w_ref[...], preferred_element_type=jnp.float32)
            + b_ref[...]
        )
        start_out(c, slot)

        @pl.when(c + NBUF < NC)
        def _():
            start_in(c + NBUF, slot)

        return carry

    jax.lax.fori_loop(0, NC, body, 0)

    for k in range(NC - NBUF, NC):
        wait_out(k, k % NBUF)


def kernel(context, sample, W_proj, b_proj):
    b2d = b_proj.reshape(1, D)
    out = pl.pallas_call(
        _mm_kernel,
        in_specs=[
            pl.BlockSpec(memory_space=pl.ANY),
            pl.BlockSpec(memory_space=pl.ANY),
            pl.BlockSpec(memory_space=pltpu.VMEM),
            pl.BlockSpec(memory_space=pltpu.VMEM),
        ],
        out_specs=pl.BlockSpec(memory_space=pl.ANY),
        out_shape=jax.ShapeDtypeStruct((2 * N_HALF, D), jnp.float32),
        scratch_shapes=[
            pltpu.VMEM((NBUF, BC, D), jnp.float32),
            pltpu.VMEM((NBUF, BC, D), jnp.float32),
            pltpu.SemaphoreType.DMA((NBUF,)),
            pltpu.SemaphoreType.DMA((NBUF,)),
        ],
    )(context, sample, W_proj, b2d)
    return out


# speedup vs baseline: 1.0288x; 1.0220x over previous
import jax
import jax.numpy as jnp
from jax.experimental import pallas as pl
from jax.experimental.pallas import tpu as pltpu

N_HALF = 25000
D = 256
BC = 5000
NCH = N_HALF // BC
NC = 2 * NCH
NBUF = 4
NSUB = 5
BS = BC // NSUB

_SCHED = [(p, j) for j in range(NCH) for p in (0, 1)]


def _mm_kernel(ctx_hbm, smp_hbm, w_ref, b_ref, out_hbm, xbuf, obuf, in_sem, out_sem):
    def in_copy(c, slot):
        src, j = _SCHED[c]
        src_ref = ctx_hbm if src == 0 else smp_hbm
        return pltpu.make_async_copy(
            src_ref.at[pl.ds(j * BC, BC), :], xbuf.at[slot], in_sem.at[slot]
        )

    def out_row(c):
        src, j = _SCHED[c]
        return src * N_HALF + j * BC

    out_copies = {}

    for s in range(NBUF):
        in_copy(s, s).start()

    for c in range(NC):
        slot = c % NBUF
        if c >= NBUF:
            for cp in out_copies.pop(c - NBUF):
                cp.wait()
        in_copy(c, slot).wait()
        if c < NC - 1:
            obuf[slot] = (
                jnp.dot(xbuf[slot], w_ref[...], preferred_element_type=jnp.float32)
                + b_ref[...]
            )
            cp = pltpu.make_async_copy(
                obuf.at[slot], out_hbm.at[pl.ds(out_row(c), BC), :], out_sem.at[slot]
            )
            cp.start()
            out_copies[c] = [cp]
        else:
            pieces = []
            for k in range(NSUB):
                obuf[slot, pl.ds(k * BS, BS), :] = (
                    jnp.dot(
                        xbuf[slot, pl.ds(k * BS, BS), :],
                        w_ref[...],
                        preferred_element_type=jnp.float32,
                    )
                    + b_ref[...]
                )
                cp = pltpu.make_async_copy(
                    obuf.at[slot, pl.ds(k * BS, BS), :],
                    out_hbm.at[pl.ds(out_row(c) + k * BS, BS), :],
                    out_sem.at[slot],
                )
                cp.start()
                pieces.append(cp)
            out_copies[c] = pieces
        if c + NBUF < NC:
            in_copy(c + NBUF, slot).start()

    for c in sorted(out_copies):
        for cp in out_copies[c]:
            cp.wait()


def kernel(context, sample, W_proj, b_proj):
    b2d = b_proj.reshape(1, D)
    out = pl.pallas_call(
        _mm_kernel,
        in_specs=[
            pl.BlockSpec(memory_space=pl.ANY),
            pl.BlockSpec(memory_space=pl.ANY),
            pl.BlockSpec(memory_space=pltpu.VMEM),
            pl.BlockSpec(memory_space=pltpu.VMEM),
        ],
        out_specs=pl.BlockSpec(memory_space=pl.ANY),
        out_shape=jax.ShapeDtypeStruct((2 * N_HALF, D), jnp.float32),
        scratch_shapes=[
            pltpu.VMEM((NBUF, BC, D), jnp.float32),
            pltpu.VMEM((NBUF, BC, D), jnp.float32),
            pltpu.SemaphoreType.DMA((NBUF,)),
            pltpu.SemaphoreType.DMA((NBUF,)),
        ],
    )(context, sample, W_proj, b2d)
    return out
